# baseline (speedup 1.0000x reference)
import jax
import jax.numpy as jnp
from jax.experimental import pallas as pl
from jax.experimental.pallas import tpu as pltpu

OUT = 256
_VMEM = 60 * 1024 * 1024
_BF = jnp.bfloat16
_F32 = jnp.float32


def _cp(*sem):
    return pltpu.CompilerParams(dimension_semantics=sem,
                                vmem_limit_bytes=_VMEM)


def _pad1(x):
    h, w, c = x.shape
    zr = jnp.zeros((1, w, c), x.dtype)
    return jnp.concatenate([zr, x, zr], axis=0)


def _conv3x3_write(xh, w2col, b, o_ref, th, row0=0):
    hp, w, c = xh.shape
    h = hp - 2
    zc = jnp.zeros((hp, 1, c), xh.dtype)
    cols = [jnp.concatenate([zc, xh[:, :-1]], axis=1),
            xh,
            jnp.concatenate([xh[:, 1:], zc], axis=1)]
    for t in range(0, h, th):
        parts = []
        for dx in range(3):
            lhs = jnp.concatenate(
                [cols[dx][t + dy:t + dy + th].reshape(th * w, c)
                 for dy in range(3)], axis=1)
            parts.append(jnp.dot(lhs, w2col[dx * 3 * c:(dx + 1) * 3 * c],
                                 preferred_element_type=_F32))
        acc = parts[0] + parts[1] + parts[2] + b
        o_ref[0, row0 + t:row0 + t + th] = (
            acc.reshape(th, w, OUT).astype(o_ref.dtype))


def _upsample(x, rh, rw):
    ho = rh.shape[0]
    wo, wi = rw.shape
    y = jnp.einsum("oh,hwc->owc", rh, x,
                   preferred_element_type=_F32)
    rwb = jnp.broadcast_to(rw, (ho, wo, wi))
    return jnp.einsum("row,rwc->roc", rwb, y,
                      preferred_element_type=_F32)


def _wup(y, rw):
    h = y.shape[0]
    wo, wi = rw.shape
    rwb = jnp.broadcast_to(rw, (h, wo, wi))
    return jnp.einsum("row,rwc->roc", rwb, y,
                      preferred_element_type=_F32)


def _k(c5_ref, c4_ref, c3h_ref, c3t_ref, c3b_ref,
       rh45_ref, rw45_ref, rh34_ref, rw34_ref,
       w51_ref, b51_ref, w52_ref, b52_ref, w6_ref, b6_ref, w7_ref, b7_ref,
       w41_ref, b41_ref, w42_ref, b42_ref, w31_ref, b31_ref, w32_ref,
       b32_ref, p3_ref, p4_ref, p5_ref, p6_ref, p7_ref,
       w51b, w52b, w6b, w7b, w41b, w42b, w31b, w32b, y_scr):
    first = (pl.program_id(0) == 0) & (pl.program_id(1) == 0)

    @pl.when(first)
    def _cast_weights():
        w51b[...] = w51_ref[...].astype(_BF)
        w52b[...] = w52_ref[...].astype(_BF)
        w6b[...] = w6_ref[...].astype(_BF)
        w7b[...] = w7_ref[...].astype(_BF)
        w41b[...] = w41_ref[...].astype(_BF)
        w42b[...] = w42_ref[...].astype(_BF)
        w31b[...] = w31_ref[...].astype(_BF)
        w32b[...] = w32_ref[...].astype(_BF)

    g = pl.program_id(1)

    @pl.when(g == 0)
    def _stage_ab():
        xb = c5_ref[0].astype(_BF)
        cin = xb.shape[-1]
        t = jnp.dot(xb.reshape(256, cin), w51b[...],
                    preferred_element_type=_F32) + b51_ref[...]
        tb = t.astype(_BF).reshape(16, 16, OUT)
        _conv3x3_write(_pad1(tb), w52b[...], b52_ref[...], p5_ref, 16)
        t6 = jnp.dot(xb.reshape(256, cin), w6b[...],
                     preferred_element_type=_F32) + b6_ref[...]
        p6v = t6.reshape(8, 2, 16, OUT)[:, 0]
        p6v = p6v.reshape(8, 8, 2, OUT)[:, :, 0]
        p6_ref[0] = p6v
        e7 = p6v.reshape(4, 2, 8, OUT)[:, 0]
        e7 = e7.reshape(4, 4, 2, OUT)[:, :, 0]
        p7 = jnp.dot(e7.reshape(16, OUT).astype(_BF), w7b[...],
                     preferred_element_type=_F32) + b7_ref[...]
        p7_ref[0] = p7.reshape(4, 4, OUT)
        up = _upsample(tb, rh45_ref[...], rw45_ref[...])
        c4b = c4_ref[0].astype(_BF)
        c4c = c4b.shape[-1]
        lat = jnp.dot(c4b.reshape(1024, c4c), w41b[...],
                      preferred_element_type=_F32) + b41_ref[...]
        s4b = (up + lat.reshape(32, 32, OUT)).astype(_BF)
        _conv3x3_write(_pad1(s4b), w42b[...], b42_ref[...], p4_ref, 16)
        y64 = jnp.einsum("oh,hwc->owc", rh34_ref[...], s4b,
                         preferred_element_type=_F32)
        y_scr[...] = y64

    w31 = w31b[...]
    w32 = w32b[...]
    rw34 = rw34_ref[...]
    c3c = c3h_ref.shape[-1]
    zrow = jnp.zeros((1, 64, OUT), _BF)

    def _stage_c(gs):
        if gs == 0:
            y33 = y_scr[0:33]
            c3rows = jnp.concatenate([c3h_ref[0], c3b_ref[0]], axis=0)
        else:
            y33 = y_scr[31:64]
            c3rows = jnp.concatenate([c3t_ref[0], c3h_ref[0]], axis=0)
        up33 = _wup(y33, rw34)
        lat = jnp.dot(c3rows.astype(_BF).reshape(33 * 64, c3c), w31,
                      preferred_element_type=_F32) + b31_ref[...]
        s3 = (up33 + lat.reshape(33, 64, OUT)).astype(_BF)
        if gs == 0:
            s3pad = jnp.concatenate([zrow, s3], axis=0)
        else:
            s3pad = jnp.concatenate([s3, zrow], axis=0)
        _conv3x3_write(s3pad, w32, b32_ref[...], p3_ref, 16)

    @pl.when(g == 0)
    def _c0():
        _stage_c(0)

    @pl.when(g == 1)
    def _c1():
        _stage_c(1)


def _full(shape):
    nd = len(shape)
    return pl.BlockSpec(shape, lambda b, g, nd=nd: (0,) * nd)


def kernel(c3_conv1_w, c3_conv1_b, c3_conv2_w, c3_conv2_b,
           c4_conv1_w, c4_conv1_b, c4_conv2_w, c4_conv2_b,
           c5_conv1_w, c5_conv1_b, c5_conv2_w, c5_conv2_b,
           c5_conv3_w, c5_conv3_b, c5_conv4_w, c5_conv4_b,
           c3, c4, c5, rh45, rw45, rh34, rw34):
    n = c5.shape[0]
    c3c, c4c, c5c_in = c3.shape[-1], c4.shape[-1], c5.shape[-1]
    w52 = c5_conv2_w.transpose(1, 0, 2, 3).reshape(9 * OUT, OUT)
    w42 = c4_conv2_w.transpose(1, 0, 2, 3).reshape(9 * OUT, OUT)
    w32 = c3_conv2_w.transpose(1, 0, 2, 3).reshape(9 * OUT, OUT)
    b = lambda v: v.reshape(1, OUT)

    p3, p4, p5, p6, p7 = pl.pallas_call(
        _k,
        grid=(n, 2),
        in_specs=[
            pl.BlockSpec((1, 16, 16, c5c_in), lambda b_, g: (b_, 0, 0, 0)),
            pl.BlockSpec((1, 32, 32, c4c), lambda b_, g: (b_, 0, 0, 0)),
            pl.BlockSpec((1, 32, 64, c3c), lambda b_, g: (b_, g, 0, 0)),
            pl.BlockSpec((1, 1, 64, c3c), lambda b_, g: (b_, 31 * g, 0, 0)),
            pl.BlockSpec((1, 1, 64, c3c),
                         lambda b_, g: (b_, 32 + 31 * g, 0, 0)),
            _full((32, 16)), _full((32, 16)),
            _full((64, 32)), _full((64, 32)),
            _full((c5c_in, OUT)), _full((1, OUT)),
            _full((9 * OUT, OUT)), _full((1, OUT)),
            _full((c5c_in, OUT)), _full((1, OUT)),
            _full((OUT, OUT)), _full((1, OUT)),
            _full((c4c, OUT)), _full((1, OUT)),
            _full((9 * OUT, OUT)), _full((1, OUT)),
            _full((c3c, OUT)), _full((1, OUT)),
            _full((9 * OUT, OUT)), _full((1, OUT)),
        ],
        out_specs=[
            pl.BlockSpec((1, 32, 64, OUT), lambda b_, g: (b_, g, 0, 0)),
            pl.BlockSpec((1, 32, 32, OUT), lambda b_, g: (b_, 0, 0, 0)),
            pl.BlockSpec((1, 16, 16, OUT), lambda b_, g: (b_, 0, 0, 0)),
            pl.BlockSpec((1, 8, 8, OUT), lambda b_, g: (b_, 0, 0, 0)),
            pl.BlockSpec((1, 4, 4, OUT), lambda b_, g: (b_, 0, 0, 0)),
        ],
        out_shape=[
            jax.ShapeDtypeStruct((n, 64, 64, OUT), _F32),
            jax.ShapeDtypeStruct((n, 32, 32, OUT), _F32),
            jax.ShapeDtypeStruct((n, 16, 16, OUT), _F32),
            jax.ShapeDtypeStruct((n, 8, 8, OUT), _F32),
            jax.ShapeDtypeStruct((n, 4, 4, OUT), _F32),
        ],
        scratch_shapes=[
            pltpu.VMEM((c5c_in, OUT), _BF), pltpu.VMEM((9 * OUT, OUT), _BF),
            pltpu.VMEM((c5c_in, OUT), _BF), pltpu.VMEM((OUT, OUT), _BF),
            pltpu.VMEM((c4c, OUT), _BF), pltpu.VMEM((9 * OUT, OUT), _BF),
            pltpu.VMEM((c3c, OUT), _BF), pltpu.VMEM((9 * OUT, OUT), _BF),
            pltpu.VMEM((64, 32, OUT), _F32)],
        compiler_params=_cp("arbitrary", "arbitrary"),
    )(c5, c4, c3, c3, c3, rh45, rw45, rh34, rw34,
      c5_conv1_w, b(c5_conv1_b), w52, b(c5_conv2_b),
      c5_conv3_w, b(c5_conv3_b), c5_conv4_w, b(c5_conv4_b),
      c4_conv1_w, b(c4_conv1_b), w42, b(c4_conv2_b),
      c3_conv1_w, b(c3_conv1_b), w32, b(c3_conv2_b))
    return p3, p4, p5, p6, p7


# speedup vs baseline: 1.0838x; 1.0838x over previous
import jax
import jax.numpy as jnp
from jax.experimental import pallas as pl
from jax.experimental.pallas import tpu as pltpu

OUT = 256
_VMEM = 60 * 1024 * 1024
_BF = jnp.bfloat16
_F32 = jnp.float32


def _cp(*sem):
    return pltpu.CompilerParams(dimension_semantics=sem,
                                vmem_limit_bytes=_VMEM)


def _padw(x):
    h, w, c = x.shape
    zc = jnp.zeros((h, 1, c), x.dtype)
    return jnp.concatenate([zc, x, zc], axis=1)


def _pad1(x):
    h, w, c = x.shape
    xp = _padw(x)
    zr = jnp.zeros((1, w + 2, c), x.dtype)
    return jnp.concatenate([zr, xp, zr], axis=0)


def _conv3x3_write(xp, w2col, b, o_ref, th, row0=0):
    hp, wp, c = xp.shape
    h, w = hp - 2, wp - 2
    cols = [xp[:, dx:dx + w, :] for dx in range(3)]
    for t in range(0, h, th):
        patches = jnp.concatenate(
            [cols[dx][t + dy:t + dy + th].reshape(th * w, c)
             for dy in range(3) for dx in range(3)], axis=1)
        acc = jnp.dot(patches, w2col,
                      preferred_element_type=_F32) + b
        o_ref[0, row0 + t:row0 + t + th] = (
            acc.reshape(th, w, OUT).astype(o_ref.dtype))


def _upsample(x, rh, rw):
    ho = rh.shape[0]
    wo, wi = rw.shape
    y = jnp.einsum("oh,hwc->owc", rh, x,
                   preferred_element_type=_F32)
    rwb = jnp.broadcast_to(rw, (ho, wo, wi))
    return jnp.einsum("row,rwc->roc", rwb, y,
                      preferred_element_type=_F32)


def _wup(y, rw):
    h = y.shape[0]
    wo, wi = rw.shape
    rwb = jnp.broadcast_to(rw, (h, wo, wi))
    return jnp.einsum("row,rwc->roc", rwb, y,
                      preferred_element_type=_F32)


def _k(c5_ref, c4_ref, c3h_ref, c3t_ref, c3b_ref,
       rh45_ref, rw45_ref, rh34_ref, rw34_ref,
       w51_ref, b51_ref, w52_ref, b52_ref, w6_ref, b6_ref, w7_ref, b7_ref,
       w41_ref, b41_ref, w42_ref, b42_ref, w31_ref, b31_ref, w32_ref,
       b32_ref, p3_ref, p4_ref, p5_ref, p6_ref, p7_ref,
       w51b, w52b, w6b, w7b, w41b, w42b, w31b, w32b, y_scr):
    first = (pl.program_id(0) == 0) & (pl.program_id(1) == 0)

    @pl.when(first)
    def _cast_weights():
        w51b[...] = w51_ref[...].astype(_BF)
        w52b[...] = w52_ref[...].astype(_BF)
        w6b[...] = w6_ref[...].astype(_BF)
        w7b[...] = w7_ref[...].astype(_BF)
        w41b[...] = w41_ref[...].astype(_BF)
        w42b[...] = w42_ref[...].astype(_BF)
        w31b[...] = w31_ref[...].astype(_BF)
        w32b[...] = w32_ref[...].astype(_BF)

    g = pl.program_id(1)

    @pl.when(g == 0)
    def _stage_ab():
        xb = c5_ref[0].astype(_BF)
        cin = xb.shape[-1]
        t = jnp.dot(xb.reshape(256, cin), w51b[...],
                    preferred_element_type=_F32) + b51_ref[...]
        tb = t.astype(_BF).reshape(16, 16, OUT)
        _conv3x3_write(_pad1(tb), w52b[...], b52_ref[...], p5_ref, 16)
        t6 = jnp.dot(xb.reshape(256, cin), w6b[...],
                     preferred_element_type=_F32) + b6_ref[...]
        p6v = t6.reshape(8, 2, 16, OUT)[:, 0]
        p6v = p6v.reshape(8, 8, 2, OUT)[:, :, 0]
        p6_ref[0] = p6v
        e7 = p6v.reshape(4, 2, 8, OUT)[:, 0]
        e7 = e7.reshape(4, 4, 2, OUT)[:, :, 0]
        p7 = jnp.dot(e7.reshape(16, OUT).astype(_BF), w7b[...],
                     preferred_element_type=_F32) + b7_ref[...]
        p7_ref[0] = p7.reshape(4, 4, OUT)
        up = _upsample(tb, rh45_ref[...], rw45_ref[...])
        c4b = c4_ref[0].astype(_BF)
        c4c = c4b.shape[-1]
        lat = jnp.dot(c4b.reshape(1024, c4c), w41b[...],
                      preferred_element_type=_F32) + b41_ref[...]
        s4b = (up + lat.reshape(32, 32, OUT)).astype(_BF)
        _conv3x3_write(_pad1(s4b), w42b[...], b42_ref[...], p4_ref, 16)
        y64 = jnp.einsum("oh,hwc->owc", rh34_ref[...], s4b,
                         preferred_element_type=_F32)
        y_scr[...] = y64

    w31 = w31b[...]
    w32 = w32b[...]
    rw34 = rw34_ref[...]
    c3c = c3h_ref.shape[-1]
    zrow = jnp.zeros((1, 64, OUT), _BF)

    def _stage_c(gs):
        if gs == 0:
            y33 = y_scr[0:33]
            c3rows = jnp.concatenate([c3h_ref[0], c3b_ref[0]], axis=0)
        else:
            y33 = y_scr[31:64]
            c3rows = jnp.concatenate([c3t_ref[0], c3h_ref[0]], axis=0)
        up33 = _wup(y33, rw34)
        lat = jnp.dot(c3rows.astype(_BF).reshape(33 * 64, c3c), w31,
                      preferred_element_type=_F32) + b31_ref[...]
        s3 = (up33 + lat.reshape(33, 64, OUT)).astype(_BF)
        if gs == 0:
            s3pad = jnp.concatenate([zrow, s3], axis=0)
        else:
            s3pad = jnp.concatenate([s3, zrow], axis=0)
        _conv3x3_write(_padw(s3pad), w32, b32_ref[...], p3_ref, 16)

    @pl.when(g == 1)
    def _c0():
        _stage_c(0)

    @pl.when(g == 2)
    def _c1():
        _stage_c(1)


def _full(shape):
    nd = len(shape)
    return pl.BlockSpec(shape, lambda b, g, nd=nd: (0,) * nd)


def kernel(c3_conv1_w, c3_conv1_b, c3_conv2_w, c3_conv2_b,
           c4_conv1_w, c4_conv1_b, c4_conv2_w, c4_conv2_b,
           c5_conv1_w, c5_conv1_b, c5_conv2_w, c5_conv2_b,
           c5_conv3_w, c5_conv3_b, c5_conv4_w, c5_conv4_b,
           c3, c4, c5, rh45, rw45, rh34, rw34):
    n = c5.shape[0]
    c3c, c4c, c5c_in = c3.shape[-1], c4.shape[-1], c5.shape[-1]
    w52 = c5_conv2_w.reshape(9 * OUT, OUT)
    w42 = c4_conv2_w.reshape(9 * OUT, OUT)
    w32 = c3_conv2_w.reshape(9 * OUT, OUT)
    b = lambda v: v.reshape(1, OUT)

    p3, p4, p5, p6, p7 = pl.pallas_call(
        _k,
        grid=(n, 3),
        in_specs=[
            pl.BlockSpec((1, 16, 16, c5c_in), lambda b_, g: (b_, 0, 0, 0)),
            pl.BlockSpec((1, 32, 32, c4c), lambda b_, g: (b_, 0, 0, 0)),
            pl.BlockSpec((1, 32, 64, c3c),
                         lambda b_, g: (b_, jnp.clip(g - 1, 0, 1), 0, 0)),
            pl.BlockSpec((1, 1, 64, c3c),
                         lambda b_, g: (b_, jnp.clip(31 * (g - 1), 0, 63), 0, 0)),
            pl.BlockSpec((1, 1, 64, c3c),
                         lambda b_, g: (b_, jnp.clip(32 + 31 * (g - 1), 0, 63), 0, 0)),
            _full((32, 16)), _full((32, 16)),
            _full((64, 32)), _full((64, 32)),
            _full((c5c_in, OUT)), _full((1, OUT)),
            _full((9 * OUT, OUT)), _full((1, OUT)),
            _full((c5c_in, OUT)), _full((1, OUT)),
            _full((OUT, OUT)), _full((1, OUT)),
            _full((c4c, OUT)), _full((1, OUT)),
            _full((9 * OUT, OUT)), _full((1, OUT)),
            _full((c3c, OUT)), _full((1, OUT)),
            _full((9 * OUT, OUT)), _full((1, OUT)),
        ],
        out_specs=[
            pl.BlockSpec((1, 32, 64, OUT),
                         lambda b_, g: (b_, jnp.clip(g - 1, 0, 1), 0, 0)),
            pl.BlockSpec((1, 32, 32, OUT), lambda b_, g: (b_, 0, 0, 0)),
            pl.BlockSpec((1, 16, 16, OUT), lambda b_, g: (b_, 0, 0, 0)),
            pl.BlockSpec((1, 8, 8, OUT), lambda b_, g: (b_, 0, 0, 0)),
            pl.BlockSpec((1, 4, 4, OUT), lambda b_, g: (b_, 0, 0, 0)),
        ],
        out_shape=[
            jax.ShapeDtypeStruct((n, 64, 64, OUT), _F32),
            jax.ShapeDtypeStruct((n, 32, 32, OUT), _F32),
            jax.ShapeDtypeStruct((n, 16, 16, OUT), _F32),
            jax.ShapeDtypeStruct((n, 8, 8, OUT), _F32),
            jax.ShapeDtypeStruct((n, 4, 4, OUT), _F32),
        ],
        scratch_shapes=[
            pltpu.VMEM((c5c_in, OUT), _BF), pltpu.VMEM((9 * OUT, OUT), _BF),
            pltpu.VMEM((c5c_in, OUT), _BF), pltpu.VMEM((OUT, OUT), _BF),
            pltpu.VMEM((c4c, OUT), _BF), pltpu.VMEM((9 * OUT, OUT), _BF),
            pltpu.VMEM((c3c, OUT), _BF), pltpu.VMEM((9 * OUT, OUT), _BF),
            pltpu.VMEM((64, 32, OUT), _F32)],
        compiler_params=_cp("arbitrary", "arbitrary"),
    )(c5, c4, c3, c3, c3, rh45, rw45, rh34, rw34,
      c5_conv1_w, b(c5_conv1_b), w52, b(c5_conv2_b),
      c5_conv3_w, b(c5_conv3_b), c5_conv4_w, b(c5_conv4_b),
      c4_conv1_w, b(c4_conv1_b), w42, b(c4_conv2_b),
      c3_conv1_w, b(c3_conv1_b), w32, b(c3_conv2_b))
    return p3, p4, p5, p6, p7


# speedup vs baseline: 1.1013x; 1.0162x over previous
import jax
import jax.numpy as jnp
from jax.experimental import pallas as pl
from jax.experimental.pallas import tpu as pltpu

OUT = 256
_VMEM = 60 * 1024 * 1024
_BF = jnp.bfloat16
_F32 = jnp.float32


def _cp(*sem):
    return pltpu.CompilerParams(dimension_semantics=sem,
                                vmem_limit_bytes=_VMEM)


def _padw(x):
    h, w, c = x.shape
    zc = jnp.zeros((h, 1, c), x.dtype)
    return jnp.concatenate([zc, x, zc], axis=1)


def _pad1(x):
    h, w, c = x.shape
    xp = _padw(x)
    zr = jnp.zeros((1, w + 2, c), x.dtype)
    return jnp.concatenate([zr, xp, zr], axis=0)


def _conv3x3_write(xp, w2col, b, o_ref, th, row0=0):
    hp, wp, c = xp.shape
    h, w = hp - 2, wp - 2
    cols = [xp[:, dx:dx + w, :] for dx in range(3)]
    for t in range(0, h, th):
        patches = jnp.concatenate(
            [cols[dx][t + dy:t + dy + th].reshape(th * w, c)
             for dy in range(3) for dx in range(3)], axis=1)
        acc = jnp.dot(patches, w2col,
                      preferred_element_type=_F32) + b
        o_ref[0, row0 + t:row0 + t + th] = (
            acc.reshape(th, w, OUT).astype(o_ref.dtype))


def _upsample(x, rh, rw):
    ho = rh.shape[0]
    wo, wi = rw.shape
    y = jnp.einsum("oh,hwc->owc", rh, x,
                   preferred_element_type=_F32)
    rwb = jnp.broadcast_to(rw, (ho, wo, wi))
    return jnp.einsum("row,rwc->roc", rwb, y,
                      preferred_element_type=_F32)


def _wup(y, rw):
    h = y.shape[0]
    wo, wi = rw.shape
    rwb = jnp.broadcast_to(rw, (h, wo, wi))
    return jnp.einsum("row,rwc->roc", rwb, y,
                      preferred_element_type=_F32)


def _k(c5_ref, c4_ref, c3h_ref, c3t_ref, c3b_ref,
       rh45_ref, rw45_ref, rh34_ref, rw34_ref,
       w51_ref, b51_ref, w52_ref, b52_ref, w6_ref, b6_ref, w7_ref, b7_ref,
       w41_ref, b41_ref, w42_ref, b42_ref, w31_ref, b31_ref, w32_ref,
       b32_ref, p3_ref, p4_ref, p5_ref, p6_ref, p7_ref,
       w51b, w52b, w6b, w7b, w41b, w42b, w31b, w32b, y_scr):
    first = (pl.program_id(0) == 0) & (pl.program_id(1) == 0)

    @pl.when(first)
    def _cast_weights():
        w51b[...] = w51_ref[...].astype(_BF)
        w52b[...] = w52_ref[...].astype(_BF)
        w6b[...] = w6_ref[...].astype(_BF)
        w7b[...] = w7_ref[...].astype(_BF)
        w41b[...] = w41_ref[...].astype(_BF)
        w42b[...] = w42_ref[...].astype(_BF)
        w31b[...] = w31_ref[...].astype(_BF)
        w32b[...] = w32_ref[...].astype(_BF)

    g = pl.program_id(1)

    @pl.when(g == 0)
    def _stage_ab():
        xb = c5_ref[0].astype(_BF)
        cin = xb.shape[-1]
        t = jnp.dot(xb.reshape(256, cin), w51b[...],
                    preferred_element_type=_F32) + b51_ref[...]
        tb = t.astype(_BF).reshape(16, 16, OUT)
        _conv3x3_write(_pad1(tb), w52b[...], b52_ref[...], p5_ref, 16)
        t6 = jnp.dot(xb.reshape(256, cin), w6b[...],
                     preferred_element_type=_F32) + b6_ref[...]
        p6v = t6.reshape(8, 2, 16, OUT)[:, 0]
        p6v = p6v.reshape(8, 8, 2, OUT)[:, :, 0]
        p6_ref[0] = p6v
        e7 = p6v.reshape(4, 2, 8, OUT)[:, 0]
        e7 = e7.reshape(4, 4, 2, OUT)[:, :, 0]
        p7 = jnp.dot(e7.reshape(16, OUT).astype(_BF), w7b[...],
                     preferred_element_type=_F32) + b7_ref[...]
        p7_ref[0] = p7.reshape(4, 4, OUT)
        up = _upsample(tb, rh45_ref[...], rw45_ref[...])
        c4b = c4_ref[0].astype(_BF)
        c4c = c4b.shape[-1]
        lat = jnp.dot(c4b.reshape(1024, c4c), w41b[...],
                      preferred_element_type=_F32) + b41_ref[...]
        s4b = (up + lat.reshape(32, 32, OUT)).astype(_BF)
        _conv3x3_write(_pad1(s4b), w42b[...], b42_ref[...], p4_ref, 32)
        y64 = jnp.einsum("oh,hwc->owc", rh34_ref[...], s4b,
                         preferred_element_type=_F32)
        y_scr[...] = y64

    w31 = w31b[...]
    w32 = w32b[...]
    rw34 = rw34_ref[...]
    c3c = c3h_ref.shape[-1]
    zrow = jnp.zeros((1, 64, OUT), _BF)

    def _stage_c(gs):
        if gs == 0:
            y33 = y_scr[0:33]
            c3rows = jnp.concatenate([c3h_ref[0], c3b_ref[0]], axis=0)
        else:
            y33 = y_scr[31:64]
            c3rows = jnp.concatenate([c3t_ref[0], c3h_ref[0]], axis=0)
        up33 = _wup(y33, rw34)
        lat = jnp.dot(c3rows.astype(_BF).reshape(33 * 64, c3c), w31,
                      preferred_element_type=_F32) + b31_ref[...]
        s3 = (up33 + lat.reshape(33, 64, OUT)).astype(_BF)
        if gs == 0:
            s3pad = jnp.concatenate([zrow, s3], axis=0)
        else:
            s3pad = jnp.concatenate([s3, zrow], axis=0)
        _conv3x3_write(_padw(s3pad), w32, b32_ref[...], p3_ref, 32)

    @pl.when(g == 0)
    def _c0():
        _stage_c(0)

    @pl.when(g == 1)
    def _c1():
        _stage_c(1)


def _full(shape):
    nd = len(shape)
    return pl.BlockSpec(shape, lambda b, g, nd=nd: (0,) * nd)


def kernel(c3_conv1_w, c3_conv1_b, c3_conv2_w, c3_conv2_b,
           c4_conv1_w, c4_conv1_b, c4_conv2_w, c4_conv2_b,
           c5_conv1_w, c5_conv1_b, c5_conv2_w, c5_conv2_b,
           c5_conv3_w, c5_conv3_b, c5_conv4_w, c5_conv4_b,
           c3, c4, c5, rh45, rw45, rh34, rw34):
    n = c5.shape[0]
    c3c, c4c, c5c_in = c3.shape[-1], c4.shape[-1], c5.shape[-1]
    w52 = c5_conv2_w.reshape(9 * OUT, OUT)
    w42 = c4_conv2_w.reshape(9 * OUT, OUT)
    w32 = c3_conv2_w.reshape(9 * OUT, OUT)
    b = lambda v: v.reshape(1, OUT)

    p3, p4, p5, p6, p7 = pl.pallas_call(
        _k,
        grid=(n, 2),
        in_specs=[
            pl.BlockSpec((1, 16, 16, c5c_in), lambda b_, g: (b_, 0, 0, 0)),
            pl.BlockSpec((1, 32, 32, c4c), lambda b_, g: (b_, 0, 0, 0)),
            pl.BlockSpec((1, 32, 64, c3c), lambda b_, g: (b_, g, 0, 0)),
            pl.BlockSpec((1, 1, 64, c3c), lambda b_, g: (b_, 31 * g, 0, 0)),
            pl.BlockSpec((1, 1, 64, c3c),
                         lambda b_, g: (b_, 32 + 31 * g, 0, 0)),
            _full((32, 16)), _full((32, 16)),
            _full((64, 32)), _full((64, 32)),
            _full((c5c_in, OUT)), _full((1, OUT)),
            _full((9 * OUT, OUT)), _full((1, OUT)),
            _full((c5c_in, OUT)), _full((1, OUT)),
            _full((OUT, OUT)), _full((1, OUT)),
            _full((c4c, OUT)), _full((1, OUT)),
            _full((9 * OUT, OUT)), _full((1, OUT)),
            _full((c3c, OUT)), _full((1, OUT)),
            _full((9 * OUT, OUT)), _full((1, OUT)),
        ],
        out_specs=[
            pl.BlockSpec((1, 32, 64, OUT), lambda b_, g: (b_, g, 0, 0)),
            pl.BlockSpec((1, 32, 32, OUT), lambda b_, g: (b_, 0, 0, 0)),
            pl.BlockSpec((1, 16, 16, OUT), lambda b_, g: (b_, 0, 0, 0)),
            pl.BlockSpec((1, 8, 8, OUT), lambda b_, g: (b_, 0, 0, 0)),
            pl.BlockSpec((1, 4, 4, OUT), lambda b_, g: (b_, 0, 0, 0)),
        ],
        out_shape=[
            jax.ShapeDtypeStruct((n, 64, 64, OUT), _F32),
            jax.ShapeDtypeStruct((n, 32, 32, OUT), _F32),
            jax.ShapeDtypeStruct((n, 16, 16, OUT), _F32),
            jax.ShapeDtypeStruct((n, 8, 8, OUT), _F32),
            jax.ShapeDtypeStruct((n, 4, 4, OUT), _F32),
        ],
        scratch_shapes=[
            pltpu.VMEM((c5c_in, OUT), _BF), pltpu.VMEM((9 * OUT, OUT), _BF),
            pltpu.VMEM((c5c_in, OUT), _BF), pltpu.VMEM((OUT, OUT), _BF),
            pltpu.VMEM((c4c, OUT), _BF), pltpu.VMEM((9 * OUT, OUT), _BF),
            pltpu.VMEM((c3c, OUT), _BF), pltpu.VMEM((9 * OUT, OUT), _BF),
            pltpu.VMEM((64, 32, OUT), _F32)],
        compiler_params=_cp("arbitrary", "arbitrary"),
    )(c5, c4, c3, c3, c3, rh45, rw45, rh34, rw34,
      c5_conv1_w, b(c5_conv1_b), w52, b(c5_conv2_b),
      c5_conv3_w, b(c5_conv3_b), c5_conv4_w, b(c5_conv4_b),
      c4_conv1_w, b(c4_conv1_b), w42, b(c4_conv2_b),
      c3_conv1_w, b(c3_conv1_b), w32, b(c3_conv2_b))
    return p3, p4, p5, p6, p7


# speedup vs baseline: 1.1059x; 1.0041x over previous
import jax
import jax.numpy as jnp
from jax.experimental import pallas as pl
from jax.experimental.pallas import tpu as pltpu

OUT = 256
_VMEM = 60 * 1024 * 1024
_BF = jnp.bfloat16
_F32 = jnp.float32


def _cp(*sem):
    return pltpu.CompilerParams(dimension_semantics=sem,
                                vmem_limit_bytes=_VMEM)


def _padw(x):
    h, w, c = x.shape
    zc = jnp.zeros((h, 1, c), x.dtype)
    return jnp.concatenate([zc, x, zc], axis=1)


def _pad1(x):
    h, w, c = x.shape
    xp = _padw(x)
    zr = jnp.zeros((1, w + 2, c), x.dtype)
    return jnp.concatenate([zr, xp, zr], axis=0)


def _conv3x3_write(xp, w2col, b, o_ref, th, row0=0):
    hp, wp, c = xp.shape
    h, w = hp - 2, wp - 2
    cols = [xp[:, dx:dx + w, :] for dx in range(3)]
    for t in range(0, h, th):
        patches = jnp.concatenate(
            [cols[dx][t + dy:t + dy + th].reshape(th * w, c)
             for dy in range(3) for dx in range(3)], axis=1)
        acc = jnp.dot(patches, w2col,
                      preferred_element_type=_F32) + b
        o_ref[0, row0 + t:row0 + t + th] = (
            acc.reshape(th, w, OUT).astype(o_ref.dtype))


def _upsample(x, rh, rw):
    ho = rh.shape[0]
    wo, wi = rw.shape
    y = jnp.einsum("oh,hwc->owc", rh, x,
                   preferred_element_type=_F32)
    rwb = jnp.broadcast_to(rw, (ho, wo, wi))
    return jnp.einsum("row,rwc->roc", rwb, y,
                      preferred_element_type=_F32)


def _wup(y, rw):
    h = y.shape[0]
    wo, wi = rw.shape
    rwb = jnp.broadcast_to(rw, (h, wo, wi))
    return jnp.einsum("row,rwc->roc", rwb, y,
                      preferred_element_type=_F32)


def _k(c5_ref, c4_ref, c3h_ref, c3t_ref, c3b_ref,
       rh45_ref, rw45_ref, rh34_ref, rw34_ref,
       w51_ref, b51_ref, w52_ref, b52_ref, w6_ref, b6_ref, w7_ref, b7_ref,
       w41_ref, b41_ref, w42_ref, b42_ref, w31_ref, b31_ref, w32_ref,
       b32_ref, p3_ref, p4_ref, p5_ref, p6_ref, p7_ref,
       w51b, w52b, w6b, w7b, w41b, w42b, w31b, w32b, y_scr):
    first = (pl.program_id(0) == 0) & (pl.program_id(1) == 0)

    @pl.when(first)
    def _cast_weights():
        w51b[...] = w51_ref[...].astype(_BF)
        w52b[...] = w52_ref[...].astype(_BF)
        w6b[...] = w6_ref[...].astype(_BF)
        w7b[...] = w7_ref[...].astype(_BF)
        w41b[...] = w41_ref[...].astype(_BF)
        w42b[...] = w42_ref[...].astype(_BF)
        w31b[...] = w31_ref[...].astype(_BF)
        w32b[...] = w32_ref[...].astype(_BF)

    g = pl.program_id(1)

    @pl.when(g == 0)
    def _stage_ab():
        xb = c5_ref[0].astype(_BF)
        cin = xb.shape[-1]
        t = jnp.dot(xb.reshape(256, cin), w51b[...],
                    preferred_element_type=_F32) + b51_ref[...]
        tb = t.astype(_BF).reshape(16, 16, OUT)
        _conv3x3_write(_pad1(tb), w52b[...], b52_ref[...], p5_ref, 16)
        t6 = jnp.dot(xb.reshape(256, cin), w6b[...],
                     preferred_element_type=_F32) + b6_ref[...]
        p6v = t6.reshape(8, 2, 16, OUT)[:, 0]
        p6v = p6v.reshape(8, 8, 2, OUT)[:, :, 0]
        p6_ref[0] = p6v
        e7 = p6v.reshape(4, 2, 8, OUT)[:, 0]
        e7 = e7.reshape(4, 4, 2, OUT)[:, :, 0]
        p7 = jnp.dot(e7.reshape(16, OUT).astype(_BF), w7b[...],
                     preferred_element_type=_F32) + b7_ref[...]
        p7_ref[0] = p7.reshape(4, 4, OUT)
        up = _upsample(tb, rh45_ref[...], rw45_ref[...])
        c4b = c4_ref[0].astype(_BF)
        c4c = c4b.shape[-1]
        lat = jnp.dot(c4b.reshape(1024, c4c), w41b[...],
                      preferred_element_type=_F32) + b41_ref[...]
        s4b = (up + lat.reshape(32, 32, OUT)).astype(_BF)
        _conv3x3_write(_pad1(s4b), w42b[...], b42_ref[...], p4_ref, 16)
        y64 = jnp.einsum("oh,hwc->owc", rh34_ref[...], s4b,
                         preferred_element_type=_F32)
        y_scr[...] = y64

    w31 = w31b[...]
    w32 = w32b[...]
    rw34 = rw34_ref[...]
    c3c = c3h_ref.shape[-1]
    zrow = jnp.zeros((1, 64, OUT), _BF)

    def _stage_c(gs):
        if gs == 0:
            y33 = y_scr[0:33]
            c3rows = jnp.concatenate([c3h_ref[0], c3b_ref[0]], axis=0)
        else:
            y33 = y_scr[31:64]
            c3rows = jnp.concatenate([c3t_ref[0], c3h_ref[0]], axis=0)
        up33 = _wup(y33, rw34)
        lat = jnp.dot(c3rows.astype(_BF).reshape(33 * 64, c3c), w31,
                      preferred_element_type=_F32) + b31_ref[...]
        s3 = (up33 + lat.reshape(33, 64, OUT)).astype(_BF)
        if gs == 0:
            s3pad = jnp.concatenate([zrow, s3], axis=0)
        else:
            s3pad = jnp.concatenate([s3, zrow], axis=0)
        _conv3x3_write(_padw(s3pad), w32, b32_ref[...], p3_ref, 16)

    @pl.when(g == 0)
    def _c0():
        _stage_c(0)

    @pl.when(g == 1)
    def _c1():
        _stage_c(1)


def _full(shape):
    nd = len(shape)
    return pl.BlockSpec(shape, lambda b, g, nd=nd: (0,) * nd)


def kernel(c3_conv1_w, c3_conv1_b, c3_conv2_w, c3_conv2_b,
           c4_conv1_w, c4_conv1_b, c4_conv2_w, c4_conv2_b,
           c5_conv1_w, c5_conv1_b, c5_conv2_w, c5_conv2_b,
           c5_conv3_w, c5_conv3_b, c5_conv4_w, c5_conv4_b,
           c3, c4, c5, rh45, rw45, rh34, rw34):
    n = c5.shape[0]
    c3c, c4c, c5c_in = c3.shape[-1], c4.shape[-1], c5.shape[-1]
    w52 = c5_conv2_w.reshape(9 * OUT, OUT)
    w42 = c4_conv2_w.reshape(9 * OUT, OUT)
    w32 = c3_conv2_w.reshape(9 * OUT, OUT)
    b = lambda v: v.reshape(1, OUT)

    p3, p4, p5, p6, p7 = pl.pallas_call(
        _k,
        grid=(n, 2),
        in_specs=[
            pl.BlockSpec((1, 16, 16, c5c_in), lambda b_, g: (b_, 0, 0, 0)),
            pl.BlockSpec((1, 32, 32, c4c), lambda b_, g: (b_, 0, 0, 0)),
            pl.BlockSpec((1, 32, 64, c3c), lambda b_, g: (b_, g, 0, 0)),
            pl.BlockSpec((1, 1, 64, c3c), lambda b_, g: (b_, 31 * g, 0, 0)),
            pl.BlockSpec((1, 1, 64, c3c),
                         lambda b_, g: (b_, 32 + 31 * g, 0, 0)),
            _full((32, 16)), _full((32, 16)),
            _full((64, 32)), _full((64, 32)),
            _full((c5c_in, OUT)), _full((1, OUT)),
            _full((9 * OUT, OUT)), _full((1, OUT)),
            _full((c5c_in, OUT)), _full((1, OUT)),
            _full((OUT, OUT)), _full((1, OUT)),
            _full((c4c, OUT)), _full((1, OUT)),
            _full((9 * OUT, OUT)), _full((1, OUT)),
            _full((c3c, OUT)), _full((1, OUT)),
            _full((9 * OUT, OUT)), _full((1, OUT)),
        ],
        out_specs=[
            pl.BlockSpec((1, 32, 64, OUT), lambda b_, g: (b_, g, 0, 0)),
            pl.BlockSpec((1, 32, 32, OUT), lambda b_, g: (b_, 0, 0, 0)),
            pl.BlockSpec((1, 16, 16, OUT), lambda b_, g: (b_, 0, 0, 0)),
            pl.BlockSpec((1, 8, 8, OUT), lambda b_, g: (b_, 0, 0, 0)),
            pl.BlockSpec((1, 4, 4, OUT), lambda b_, g: (b_, 0, 0, 0)),
        ],
        out_shape=[
            jax.ShapeDtypeStruct((n, 64, 64, OUT), _F32),
            jax.ShapeDtypeStruct((n, 32, 32, OUT), _F32),
            jax.ShapeDtypeStruct((n, 16, 16, OUT), _F32),
            jax.ShapeDtypeStruct((n, 8, 8, OUT), _F32),
            jax.ShapeDtypeStruct((n, 4, 4, OUT), _F32),
        ],
        scratch_shapes=[
            pltpu.VMEM((c5c_in, OUT), _BF), pltpu.VMEM((9 * OUT, OUT), _BF),
            pltpu.VMEM((c5c_in, OUT), _BF), pltpu.VMEM((OUT, OUT), _BF),
            pltpu.VMEM((c4c, OUT), _BF), pltpu.VMEM((9 * OUT, OUT), _BF),
            pltpu.VMEM((c3c, OUT), _BF), pltpu.VMEM((9 * OUT, OUT), _BF),
            pltpu.VMEM((64, 32, OUT), _F32)],
        compiler_params=_cp("arbitrary", "arbitrary"),
    )(c5, c4, c3, c3, c3, rh45, rw45, rh34, rw34,
      c5_conv1_w, b(c5_conv1_b), w52, b(c5_conv2_b),
      c5_conv3_w, b(c5_conv3_b), c5_conv4_w, b(c5_conv4_b),
      c4_conv1_w, b(c4_conv1_b), w42, b(c4_conv2_b),
      c3_conv1_w, b(c3_conv1_b), w32, b(c3_conv2_b))
    return p3, p4, p5, p6, p7
